# baseline (speedup 1.0000x reference)
import functools

import jax
import jax.numpy as jnp
from jax import lax
from jax.experimental import pallas as pl
from jax.experimental.pallas import tpu as pltpu
from jax.experimental.pallas import tpu_sc as plsc

E = 160000
N = 10000
H = 128
EF = 16
NB = 6
NMOL = 100
MOLSZ = 100

NC = 2
NS = 16
NW = NC * NS
EPW = E // NW
CE = 40
NCHUNK = EPW // CE
RN = 40
NODE_CHUNKS = N // RN
RB = 640
F32 = jnp.float32


def _mesh():
    return plsc.VectorSubcoreMesh(
        core_axis_name="c", subcore_axis_name="s", num_cores=NC, num_subcores=NS
    )


def _wid():
    return lax.axis_index("s") * NC + lax.axis_index("c")


def _sigmoid16(x):
    return 1.0 / (1.0 + jnp.exp(-x))


CG = 128
NCG = EPW // CG
TAILG = EPW - NCG * CG


@functools.cache
def _sc_gather_rows():
    @functools.partial(
        pl.kernel,
        out_type=jax.ShapeDtypeStruct((E, H), F32),
        mesh=_mesh(),
        scratch_types=[
            pltpu.VMEM((EPW,), jnp.int32),
            pltpu.VMEM((3, CG, H), F32),
            pltpu.SemaphoreType.DMA,
            pltpu.SemaphoreType.DMA,
            pltpu.SemaphoreType.DMA,
            pltpu.SemaphoreType.DMA,
            pltpu.SemaphoreType.DMA,
            pltpu.SemaphoreType.DMA,
        ],
    )
    def k(table_hbm, idx_hbm, out_hbm, idx_all, rows, g0, g1, g2, o0, o1, o2):
        gsems = (g0, g1, g2)
        osems = (o0, o1, o2)
        w0 = _wid() * EPW
        pltpu.sync_copy(idx_hbm.at[pl.ds(w0, EPW)], idx_all)

        def gcp(ci, b, n):
            off = pl.multiple_of(ci * CG, 8)
            return pltpu.make_async_copy(
                table_hbm.at[idx_all.at[pl.ds(off, n)]],
                rows.at[b, pl.ds(0, n)], gsems[b])

        def ow_start(ci, b, n):
            off = pl.multiple_of(ci * CG, 8)
            pltpu.async_copy(rows.at[b, pl.ds(0, n)],
                             out_hbm.at[pl.ds(w0 + off, n)], osems[b])

        def ow_wait(b, n):
            pltpu.make_async_copy(rows.at[b, pl.ds(0, n)],
                                  out_hbm.at[pl.ds(0, n)], osems[b]).wait()

        gcp(0, 0, CG).start()

        def outer(i, carry):
            for b in range(3):
                ci = i * 3 + b
                gcp(ci, b, CG).wait()
                nb = (b + 1) % 3

                @pl.when(ci + 1 < NCG)
                def _():
                    @pl.when(ci >= 2)
                    def _():
                        ow_wait(nb, CG)

                    gcp(ci + 1, nb, CG).start()

                ow_start(ci, b, CG)
            return carry

        lax.fori_loop(0, NCG // 3, outer, 0)
        ow_wait(1, CG)
        ow_wait(2, CG)
        ow_wait(0, CG)

        gcp(NCG, 0, TAILG).start()
        gcp(NCG, 0, TAILG).wait()
        ow_start(NCG, 0, TAILG)
        ow_wait(0, TAILG)

    return k


CE2 = 24

S1 = 81920
S2 = E - S1
PW1 = S1 // NW
PW2 = S2 // NW


@functools.cache
def _sc_msg(lo, npw):
    nch = npw // CE2
    tail = npw - nch * CE2
    rows = NW * npw

    @functools.partial(
        pl.kernel,
        out_type=jax.ShapeDtypeStruct((rows, 2 * H), F32),
        mesh=_mesh(),
        scratch_types=[
            pltpu.VMEM((NB * npw,), jnp.int32),
            pltpu.VMEM((2, NB, CE2, 2 * H), F32),
            pltpu.VMEM((2, CE2, H), F32),
            pltpu.VMEM((2, CE2, 2 * H), F32),
            pltpu.SemaphoreType.DMA,
            pltpu.SemaphoreType.DMA,
            pltpu.SemaphoreType.DMA,
            pltpu.SemaphoreType.DMA,
        ],
    )
    def k(tab_hbm, rm_hbm, bgT_hbm, out_hbm, idx_all, gb, rmv, ob,
          gs0, gs1, os0, os1):
        gsems = (gs0, gs1)
        osems = (os0, os1)
        w0l = _wid() * npw
        w0g = lo + w0l
        for j in range(NB):
            pltpu.sync_copy(bgT_hbm.at[pl.ds(j * E + w0g, npw)],
                            idx_all.at[pl.ds(j * npw, npw)])

        def gather_cps(ci, b, n):
            off = pl.multiple_of(ci * CE2, 8)
            cps = [
                pltpu.make_async_copy(
                    tab_hbm.at[idx_all.at[pl.ds(j * npw + off, n)]],
                    gb.at[b, j, pl.ds(0, n)], gsems[b])
                for j in range(NB)
            ]
            cps.append(pltpu.make_async_copy(
                rm_hbm.at[pl.ds(w0g + off, n)],
                rmv.at[b, pl.ds(0, n)], gsems[b]))
            return cps

        def owrite_start(ci, b, n):
            off = pl.multiple_of(ci * CE2, 8)
            pltpu.async_copy(ob.at[b, pl.ds(0, n)],
                             out_hbm.at[pl.ds(w0l + off, n)], osems[b])

        def owrite_wait(b, n):
            pltpu.make_async_copy(ob.at[b, pl.ds(0, n)],
                                  out_hbm.at[pl.ds(0, n)], osems[b]).wait()

        def compute(b, n):
            @plsc.parallel_loop(0, n)
            def edge(e):
                for sl in range(H // 16):
                    o = sl * 16
                    rv = rmv[b, e, pl.ds(o, 16)]
                    accs = jnp.zeros((16,), F32)
                    accg = jnp.zeros((16,), F32)
                    for j in range(NB):
                        hv = gb[b, j, e, pl.ds(o, 16)]
                        uv = gb[b, j, e, pl.ds(H + o, 16)]
                        g = 1.0 / (1.0 + jnp.exp(rv + uv))
                        accs = accs + hv
                        accg = accg + g * hv
                    ob[b, e, pl.ds(o, 16)] = accs
                    ob[b, e, pl.ds(H + o, 16)] = accg

        for cp in gather_cps(0, 0, CE2):
            cp.start()

        def outer(i, carry):
            for b in range(2):
                ci = i * 2 + b
                for cp in gather_cps(ci, b, CE2):
                    cp.wait()

                @pl.when(ci + 1 < nch)
                def _():
                    for cp in gather_cps(ci + 1, 1 - b, CE2):
                        cp.start()

                @pl.when(ci >= 2)
                def _():
                    owrite_wait(b, CE2)

                compute(b, CE2)
                owrite_start(ci, b, CE2)
            return carry

        lax.fori_loop(0, nch // 2, outer, 0)

        if nch % 2 == 1:
            ci = nch - 1
            for cp in gather_cps(ci, 0, CE2):
                cp.wait()
            owrite_wait(0, CE2)
            compute(0, CE2)
            owrite_start(ci, 0, CE2)
            tb = 1
        else:
            tb = 0

        if tail:
            owrite_wait(tb, CE2)
            for cp in gather_cps(nch, tb, tail):
                cp.start()
            for cp in gather_cps(nch, tb, tail):
                cp.wait()
            compute(tb, tail)
            owrite_start(nch, tb, tail)
            owrite_wait(tb, tail)
        owrite_wait(1 - tb, CE2)

    return k


@functools.cache
def _sc_nbr():
    @functools.partial(
        pl.kernel,
        out_type=jax.ShapeDtypeStruct((N, H), F32),
        mesh=_mesh(),
        scratch_types=[
            pltpu.VMEM((NB, RN), jnp.int32),
            pltpu.VMEM((NB, RN, H), F32),
            pltpu.VMEM((RN, H), F32),
            pltpu.SemaphoreType.DMA,
        ],
    )
    def k(h_hbm, agT_hbm, out_hbm, idx_v, gb_v, ob_v, sem):
        w = _wid()
        steps = (NODE_CHUNKS + NW - 1) // NW

        def step(si, carry):
            ci = w + si * NW

            @pl.when(ci < NODE_CHUNKS)
            def _():
                base = pl.multiple_of(ci * RN, 8)
                for j in range(NB):
                    pltpu.sync_copy(agT_hbm.at[pl.ds(j * N + base, RN)],
                                    idx_v.at[j])
                cps = [
                    pltpu.async_copy(h_hbm.at[idx_v.at[j]], gb_v.at[j], sem)
                    for j in range(NB)
                ]
                for cp in cps:
                    cp.wait()

                def node(e, ecarry):
                    for sl in range(H // 16):
                        o = sl * 16
                        acc = jnp.zeros((16,), F32)
                        for j in range(NB):
                            acc = acc + gb_v[j, e, pl.ds(o, 16)]
                        ob_v[e, pl.ds(o, 16)] = acc
                    return ecarry

                lax.fori_loop(0, RN, node, 0)
                pltpu.sync_copy(ob_v, out_hbm.at[pl.ds(base, RN)])

            return carry

        lax.fori_loop(0, steps, step, 0)

    return k


def _dot(a, b):
    return jnp.dot(a, b, preferred_element_type=F32)


def _mask_row0(x, enable=True):
    if not enable:
        return x
    rows = lax.broadcasted_iota(jnp.int32, x.shape, 0)
    first = pl.program_id(0) == 0
    return jnp.where(jnp.logical_and(rows == 0, first), 0.0, x)


def _tc_pre_body(fs_ref, ef_ref, wpre, bz, bh, urT,
                 pz_ref, rm_ref, ph_ref, tab_ref):
    X = jnp.concatenate([fs_ref[...], ef_ref[...]], axis=1)
    P3 = _dot(X, wpre[...])
    pz = P3[:, :H] + bz[...]
    ph = P3[:, 2 * H:] + bh[...]
    pz_ref[...] = pz
    rm_ref[...] = P3[:, H:2 * H]
    ph_ref[...] = ph
    h1 = jax.nn.sigmoid(pz) * jnp.tanh(ph)
    h1 = _mask_row0(h1)
    tab_ref[:, :H] = h1
    tab_ref[:, H:] = _dot(h1, -urT[...])


@functools.cache
def _tc_pre():
    rspec = lambda w: pl.BlockSpec((RB, w), lambda i: (i, 0))
    wspec = pl.BlockSpec((H, H), lambda i: (0, 0))
    pspec = pl.BlockSpec((H + EF, 3 * H), lambda i: (0, 0))
    bspec = pl.BlockSpec((1, H), lambda i: (0, 0))
    return pl.pallas_call(
        _tc_pre_body,
        grid=(E // RB,),
        in_specs=[rspec(H), rspec(EF), pspec, bspec, bspec, wspec],
        out_specs=[rspec(H), rspec(H), rspec(H), rspec(2 * H)],
        out_shape=[jax.ShapeDtypeStruct((E, H), F32)] * 3
        + [jax.ShapeDtypeStruct((E, 2 * H), F32)],
    )


def _tc_gru_body(sum_ref, pz_ref, ph_ref, wzp, urT, out_ref, *, last,
                 mask):
    s_h = sum_ref[:, :H]
    zp = _dot(sum_ref[...], wzp[...])
    z = jax.nn.sigmoid(pz_ref[...] + zp[:, :H])
    p = jnp.tanh(ph_ref[...] + zp[:, H:])
    h = (1.0 - z) * s_h + z * p
    h = _mask_row0(h, mask)
    if last:
        out_ref[...] = h
    else:
        out_ref[:, :H] = h
        out_ref[:, H:] = _dot(h, -urT[...])


@functools.cache
def _tc_gru(last, lo, rows):
    blk0 = lo // RB
    rspec = lambda w: pl.BlockSpec((RB, w), lambda i: (i, 0))
    gspec = lambda w: pl.BlockSpec((RB, w), lambda i: (i + blk0, 0))
    wspec = pl.BlockSpec((H, H), lambda i: (0, 0))
    zspec = pl.BlockSpec((2 * H, 2 * H), lambda i: (0, 0))
    ow = H if last else 2 * H
    specs = [rspec(2 * H), gspec(H), gspec(H), zspec, wspec]
    return pl.pallas_call(
        functools.partial(_tc_gru_body, last=last, mask=(lo == 0)),
        grid=(rows // RB,),
        in_specs=specs,
        out_specs=rspec(ow),
        out_shape=jax.ShapeDtypeStruct((rows, ow), F32),
    )


def _tc_out_body(fn_ref, nei_ref, wo12, bo, hatom_ref, hmol_ref):
    X = jnp.concatenate([fn_ref[0], nei_ref[0]], axis=1)
    x = _dot(X, wo12[...]) + bo[...]
    x = jnp.maximum(x, 0.0)
    x = _mask_row0(x)
    hatom_ref[0] = x
    hmol_ref[0] = jnp.sum(x, axis=0, keepdims=True)


@functools.cache
def _tc_out():
    rspec = pl.BlockSpec((1, MOLSZ, H), lambda i: (i, 0, 0))
    wspec = pl.BlockSpec((2 * H, H), lambda i: (0, 0))
    bspec = pl.BlockSpec((1, H), lambda i: (0, 0))
    return pl.pallas_call(
        _tc_out_body,
        grid=(NMOL,),
        in_specs=[rspec, rspec, wspec, bspec],
        out_specs=[rspec, pl.BlockSpec((1, 1, H), lambda i: (i, 0, 0))],
        out_shape=[jax.ShapeDtypeStruct((NMOL, MOLSZ, H), F32),
                   jax.ShapeDtypeStruct((NMOL, 1, H), F32)],
    )


def kernel(fnode, fmess, agraph, bgraph, atom_scope, W_z, b_z, W_r, U_r,
           W_h, b_h, W_o, b_o):
    src = fmess[:, 0].astype(jnp.int32)
    efeat = fmess[:, 2:]
    bgT = bgraph.T.reshape(-1)
    agT = agraph.T.reshape(-1)

    wz2 = W_z[:, H + EF:].T
    wh2 = W_h[:, H + EF:].T
    wpre = jnp.concatenate(
        [W_z[:, :H + EF].T, -W_r.T, W_h[:, :H + EF].T], axis=1)
    zeroH = jnp.zeros((H, H), F32)
    wzp = jnp.concatenate(
        [jnp.concatenate([wz2, zeroH], axis=1),
         jnp.concatenate([zeroH, wh2], axis=1)], axis=0)
    wo12 = W_o.T
    urT = U_r.T
    bz = b_z.reshape(1, H)
    bh = b_h.reshape(1, H)
    bo = b_o.reshape(1, H)

    fsrc = _sc_gather_rows()(fnode, src)
    pz, rm, ph, tab = _tc_pre()(fsrc, efeat, wpre, bz, bh, urT)
    sums = _sc_msg(0, EPW)(tab, rm, bgT)
    tab = _tc_gru(False, 0, E)(sums, pz, ph, wzp, urT)
    sums = _sc_msg(0, EPW)(tab, rm, bgT)
    h = _tc_gru(True, 0, E)(sums, pz, ph, wzp, urT)
    nei = _sc_nbr()(h, agT)
    hatom3, hmol3 = _tc_out()(fnode.reshape(NMOL, MOLSZ, H),
                              nei.reshape(NMOL, MOLSZ, H), wo12, bo)
    return (hmol3.reshape(NMOL, H), hatom3.reshape(N, H))


# speedup vs baseline: 1.0487x; 1.0487x over previous
import functools

import jax
import jax.numpy as jnp
from jax import lax
from jax.experimental import pallas as pl
from jax.experimental.pallas import tpu as pltpu
from jax.experimental.pallas import tpu_sc as plsc

E = 160000
N = 10000
H = 128
EF = 16
NB = 6
NMOL = 100
MOLSZ = 100

NC = 2
NS = 16
NW = NC * NS
EPW = E // NW
CE = 40
NCHUNK = EPW // CE
RN = 40
NODE_CHUNKS = N // RN
RB = 800
F32 = jnp.float32


def _mesh():
    return plsc.VectorSubcoreMesh(
        core_axis_name="c", subcore_axis_name="s", num_cores=NC, num_subcores=NS
    )


def _wid():
    return lax.axis_index("s") * NC + lax.axis_index("c")


def _sigmoid16(x):
    return 1.0 / (1.0 + jnp.exp(-x))


CG = 128
NCG = EPW // CG
TAILG = EPW - NCG * CG


@functools.cache
def _sc_gather_rows():
    @functools.partial(
        pl.kernel,
        out_type=jax.ShapeDtypeStruct((E, H), F32),
        mesh=_mesh(),
        scratch_types=[
            pltpu.VMEM((EPW,), jnp.int32),
            pltpu.VMEM((3, CG, H), F32),
            pltpu.SemaphoreType.DMA,
            pltpu.SemaphoreType.DMA,
            pltpu.SemaphoreType.DMA,
            pltpu.SemaphoreType.DMA,
            pltpu.SemaphoreType.DMA,
            pltpu.SemaphoreType.DMA,
        ],
    )
    def k(table_hbm, idx_hbm, out_hbm, idx_all, rows, g0, g1, g2, o0, o1, o2):
        gsems = (g0, g1, g2)
        osems = (o0, o1, o2)
        w0 = _wid() * EPW
        pltpu.sync_copy(idx_hbm.at[pl.ds(w0, EPW)], idx_all)

        def gcp(ci, b, n):
            off = pl.multiple_of(ci * CG, 8)
            return pltpu.make_async_copy(
                table_hbm.at[idx_all.at[pl.ds(off, n)]],
                rows.at[b, pl.ds(0, n)], gsems[b])

        def ow_start(ci, b, n):
            off = pl.multiple_of(ci * CG, 8)
            pltpu.async_copy(rows.at[b, pl.ds(0, n)],
                             out_hbm.at[pl.ds(w0 + off, n)], osems[b])

        def ow_wait(b, n):
            pltpu.make_async_copy(rows.at[b, pl.ds(0, n)],
                                  out_hbm.at[pl.ds(0, n)], osems[b]).wait()

        gcp(0, 0, CG).start()

        def outer(i, carry):
            for b in range(3):
                ci = i * 3 + b
                gcp(ci, b, CG).wait()
                nb = (b + 1) % 3

                @pl.when(ci + 1 < NCG)
                def _():
                    @pl.when(ci >= 2)
                    def _():
                        ow_wait(nb, CG)

                    gcp(ci + 1, nb, CG).start()

                ow_start(ci, b, CG)
            return carry

        lax.fori_loop(0, NCG // 3, outer, 0)
        ow_wait(1, CG)
        ow_wait(2, CG)
        ow_wait(0, CG)

        gcp(NCG, 0, TAILG).start()
        gcp(NCG, 0, TAILG).wait()
        ow_start(NCG, 0, TAILG)
        ow_wait(0, TAILG)

    return k


CE2 = 24

S1 = 81920
S2 = E - S1
PW1 = S1 // NW
PW2 = S2 // NW


@functools.cache
def _sc_msg(lo, npw):
    nch = npw // CE2
    tail = npw - nch * CE2
    rows = NW * npw

    @functools.partial(
        pl.kernel,
        out_type=jax.ShapeDtypeStruct((rows, 2 * H), F32),
        mesh=_mesh(),
        scratch_types=[
            pltpu.VMEM((NB * npw,), jnp.int32),
            pltpu.VMEM((2, NB, CE2, 2 * H), F32),
            pltpu.VMEM((2, CE2, H), F32),
            pltpu.VMEM((2, CE2, 2 * H), F32),
            pltpu.SemaphoreType.DMA,
            pltpu.SemaphoreType.DMA,
            pltpu.SemaphoreType.DMA,
            pltpu.SemaphoreType.DMA,
        ],
    )
    def k(tab_hbm, rm_hbm, bgT_hbm, out_hbm, idx_all, gb, rmv, ob,
          gs0, gs1, os0, os1):
        gsems = (gs0, gs1)
        osems = (os0, os1)
        w0l = _wid() * npw
        w0g = lo + w0l
        for j in range(NB):
            pltpu.sync_copy(bgT_hbm.at[pl.ds(j * E + w0g, npw)],
                            idx_all.at[pl.ds(j * npw, npw)])

        def gather_cps(ci, b, n):
            off = pl.multiple_of(ci * CE2, 8)
            cps = [
                pltpu.make_async_copy(
                    tab_hbm.at[idx_all.at[pl.ds(j * npw + off, n)]],
                    gb.at[b, j, pl.ds(0, n)], gsems[b])
                for j in range(NB)
            ]
            cps.append(pltpu.make_async_copy(
                rm_hbm.at[pl.ds(w0g + off, n)],
                rmv.at[b, pl.ds(0, n)], gsems[b]))
            return cps

        def owrite_start(ci, b, n):
            off = pl.multiple_of(ci * CE2, 8)
            pltpu.async_copy(ob.at[b, pl.ds(0, n)],
                             out_hbm.at[pl.ds(w0l + off, n)], osems[b])

        def owrite_wait(b, n):
            pltpu.make_async_copy(ob.at[b, pl.ds(0, n)],
                                  out_hbm.at[pl.ds(0, n)], osems[b]).wait()

        def compute(b, n):
            @plsc.parallel_loop(0, n)
            def edge(e):
                for sl in range(H // 16):
                    o = sl * 16
                    rv = rmv[b, e, pl.ds(o, 16)]
                    accs = jnp.zeros((16,), F32)
                    accg = jnp.zeros((16,), F32)
                    for j in range(NB):
                        hv = gb[b, j, e, pl.ds(o, 16)]
                        uv = gb[b, j, e, pl.ds(H + o, 16)]
                        g = 1.0 / (1.0 + jnp.exp(rv + uv))
                        accs = accs + hv
                        accg = accg + g * hv
                    ob[b, e, pl.ds(o, 16)] = accs
                    ob[b, e, pl.ds(H + o, 16)] = accg

        for cp in gather_cps(0, 0, CE2):
            cp.start()

        def outer(i, carry):
            for b in range(2):
                ci = i * 2 + b
                for cp in gather_cps(ci, b, CE2):
                    cp.wait()

                @pl.when(ci + 1 < nch)
                def _():
                    for cp in gather_cps(ci + 1, 1 - b, CE2):
                        cp.start()

                @pl.when(ci >= 2)
                def _():
                    owrite_wait(b, CE2)

                compute(b, CE2)
                owrite_start(ci, b, CE2)
            return carry

        lax.fori_loop(0, nch // 2, outer, 0)

        if nch % 2 == 1:
            ci = nch - 1
            for cp in gather_cps(ci, 0, CE2):
                cp.wait()
            owrite_wait(0, CE2)
            compute(0, CE2)
            owrite_start(ci, 0, CE2)
            tb = 1
        else:
            tb = 0

        if tail:
            owrite_wait(tb, CE2)
            for cp in gather_cps(nch, tb, tail):
                cp.start()
            for cp in gather_cps(nch, tb, tail):
                cp.wait()
            compute(tb, tail)
            owrite_start(nch, tb, tail)
            owrite_wait(tb, tail)
        owrite_wait(1 - tb, CE2)

    return k


@functools.cache
def _sc_nbr():
    @functools.partial(
        pl.kernel,
        out_type=jax.ShapeDtypeStruct((N, H), F32),
        mesh=_mesh(),
        scratch_types=[
            pltpu.VMEM((NB, RN), jnp.int32),
            pltpu.VMEM((NB, RN, H), F32),
            pltpu.VMEM((RN, H), F32),
            pltpu.SemaphoreType.DMA,
        ],
    )
    def k(h_hbm, agT_hbm, out_hbm, idx_v, gb_v, ob_v, sem):
        w = _wid()
        steps = (NODE_CHUNKS + NW - 1) // NW

        def step(si, carry):
            ci = w + si * NW

            @pl.when(ci < NODE_CHUNKS)
            def _():
                base = pl.multiple_of(ci * RN, 8)
                for j in range(NB):
                    pltpu.sync_copy(agT_hbm.at[pl.ds(j * N + base, RN)],
                                    idx_v.at[j])
                cps = [
                    pltpu.async_copy(h_hbm.at[idx_v.at[j]], gb_v.at[j], sem)
                    for j in range(NB)
                ]
                for cp in cps:
                    cp.wait()

                def node(e, ecarry):
                    for sl in range(H // 16):
                        o = sl * 16
                        acc = jnp.zeros((16,), F32)
                        for j in range(NB):
                            acc = acc + gb_v[j, e, pl.ds(o, 16)]
                        ob_v[e, pl.ds(o, 16)] = acc
                    return ecarry

                lax.fori_loop(0, RN, node, 0)
                pltpu.sync_copy(ob_v, out_hbm.at[pl.ds(base, RN)])

            return carry

        lax.fori_loop(0, steps, step, 0)

    return k


def _dot(a, b):
    return jnp.dot(a, b, preferred_element_type=F32)


def _mask_row0(x, enable=True):
    if not enable:
        return x
    rows = lax.broadcasted_iota(jnp.int32, x.shape, 0)
    first = pl.program_id(0) == 0
    return jnp.where(jnp.logical_and(rows == 0, first), 0.0, x)


def _tc_pre_body(fs_ref, ef_ref, wpre, bz, bh, urT,
                 pz_ref, rm_ref, ph_ref, tab_ref):
    X = jnp.concatenate([fs_ref[...], ef_ref[...]], axis=1)
    P3 = _dot(X, wpre[...])
    pz = P3[:, :H] + bz[...]
    ph = P3[:, 2 * H:] + bh[...]
    pz_ref[...] = pz
    rm_ref[...] = P3[:, H:2 * H]
    ph_ref[...] = ph
    h1 = jax.nn.sigmoid(pz) * jnp.tanh(ph)
    h1 = _mask_row0(h1)
    tab_ref[:, :H] = h1
    tab_ref[:, H:] = _dot(h1, -urT[...])


@functools.cache
def _tc_pre():
    rspec = lambda w: pl.BlockSpec((RB, w), lambda i: (i, 0))
    wspec = pl.BlockSpec((H, H), lambda i: (0, 0))
    pspec = pl.BlockSpec((H + EF, 3 * H), lambda i: (0, 0))
    bspec = pl.BlockSpec((1, H), lambda i: (0, 0))
    return pl.pallas_call(
        _tc_pre_body,
        grid=(E // RB,),
        in_specs=[rspec(H), rspec(EF), pspec, bspec, bspec, wspec],
        out_specs=[rspec(H), rspec(H), rspec(H), rspec(2 * H)],
        out_shape=[jax.ShapeDtypeStruct((E, H), F32)] * 3
        + [jax.ShapeDtypeStruct((E, 2 * H), F32)],
    )


def _tc_gru_body(sum_ref, pz_ref, ph_ref, wzp, urT, out_ref, *, last,
                 mask):
    s_h = sum_ref[:, :H]
    zp = _dot(sum_ref[...], wzp[...])
    z = jax.nn.sigmoid(pz_ref[...] + zp[:, :H])
    p = jnp.tanh(ph_ref[...] + zp[:, H:])
    h = (1.0 - z) * s_h + z * p
    h = _mask_row0(h, mask)
    if last:
        out_ref[...] = h
    else:
        out_ref[:, :H] = h
        out_ref[:, H:] = _dot(h, -urT[...])


@functools.cache
def _tc_gru(last, lo, rows):
    blk0 = lo // RB
    rspec = lambda w: pl.BlockSpec((RB, w), lambda i: (i, 0))
    gspec = lambda w: pl.BlockSpec((RB, w), lambda i: (i + blk0, 0))
    wspec = pl.BlockSpec((H, H), lambda i: (0, 0))
    zspec = pl.BlockSpec((2 * H, 2 * H), lambda i: (0, 0))
    ow = H if last else 2 * H
    specs = [rspec(2 * H), gspec(H), gspec(H), zspec, wspec]
    return pl.pallas_call(
        functools.partial(_tc_gru_body, last=last, mask=(lo == 0)),
        grid=(rows // RB,),
        in_specs=specs,
        out_specs=rspec(ow),
        out_shape=jax.ShapeDtypeStruct((rows, ow), F32),
    )


def _tc_out_body(fn_ref, nei_ref, wo12, bo, hatom_ref, hmol_ref):
    X = jnp.concatenate([fn_ref[0], nei_ref[0]], axis=1)
    x = _dot(X, wo12[...]) + bo[...]
    x = jnp.maximum(x, 0.0)
    x = _mask_row0(x)
    hatom_ref[0] = x
    hmol_ref[0] = jnp.sum(x, axis=0, keepdims=True)


@functools.cache
def _tc_out():
    rspec = pl.BlockSpec((1, MOLSZ, H), lambda i: (i, 0, 0))
    wspec = pl.BlockSpec((2 * H, H), lambda i: (0, 0))
    bspec = pl.BlockSpec((1, H), lambda i: (0, 0))
    return pl.pallas_call(
        _tc_out_body,
        grid=(NMOL,),
        in_specs=[rspec, rspec, wspec, bspec],
        out_specs=[rspec, pl.BlockSpec((1, 1, H), lambda i: (i, 0, 0))],
        out_shape=[jax.ShapeDtypeStruct((NMOL, MOLSZ, H), F32),
                   jax.ShapeDtypeStruct((NMOL, 1, H), F32)],
    )


def kernel(fnode, fmess, agraph, bgraph, atom_scope, W_z, b_z, W_r, U_r,
           W_h, b_h, W_o, b_o):
    src = fmess[:, 0].astype(jnp.int32)
    efeat = fmess[:, 2:]
    bgT = bgraph.T.reshape(-1)
    agT = agraph.T.reshape(-1)

    wz2 = W_z[:, H + EF:].T
    wh2 = W_h[:, H + EF:].T
    wpre = jnp.concatenate(
        [W_z[:, :H + EF].T, -W_r.T, W_h[:, :H + EF].T], axis=1)
    zeroH = jnp.zeros((H, H), F32)
    wzp = jnp.concatenate(
        [jnp.concatenate([wz2, zeroH], axis=1),
         jnp.concatenate([zeroH, wh2], axis=1)], axis=0)
    wo12 = W_o.T
    urT = U_r.T
    bz = b_z.reshape(1, H)
    bh = b_h.reshape(1, H)
    bo = b_o.reshape(1, H)

    fsrc = _sc_gather_rows()(fnode, src)
    pz, rm, ph, tab = _tc_pre()(fsrc, efeat, wpre, bz, bh, urT)
    sums = _sc_msg(0, EPW)(tab, rm, bgT)
    tab = _tc_gru(False, 0, E)(sums, pz, ph, wzp, urT)
    sums = _sc_msg(0, EPW)(tab, rm, bgT)
    h = _tc_gru(True, 0, E)(sums, pz, ph, wzp, urT)
    nei = _sc_nbr()(h, agT)
    hatom3, hmol3 = _tc_out()(fnode.reshape(NMOL, MOLSZ, H),
                              nei.reshape(NMOL, MOLSZ, H), wo12, bo)
    return (hmol3.reshape(NMOL, H), hatom3.reshape(N, H))


# speedup vs baseline: 1.1643x; 1.1102x over previous
import functools

import jax
import jax.numpy as jnp
from jax import lax
from jax.experimental import pallas as pl
from jax.experimental.pallas import tpu as pltpu
from jax.experimental.pallas import tpu_sc as plsc

E = 160000
N = 10000
H = 128
EF = 16
NB = 6
NMOL = 100
MOLSZ = 100

NC = 2
NS = 16
NW = NC * NS
EPW = E // NW
CE = 40
NCHUNK = EPW // CE
RN = 40
NODE_CHUNKS = N // RN
RB = 1600
F32 = jnp.float32


def _mesh():
    return plsc.VectorSubcoreMesh(
        core_axis_name="c", subcore_axis_name="s", num_cores=NC, num_subcores=NS
    )


def _wid():
    return lax.axis_index("s") * NC + lax.axis_index("c")


def _sigmoid16(x):
    return 1.0 / (1.0 + jnp.exp(-x))


CG = 128
NCG = EPW // CG
TAILG = EPW - NCG * CG


@functools.cache
def _sc_gather_rows():
    @functools.partial(
        pl.kernel,
        out_type=jax.ShapeDtypeStruct((E, H), F32),
        mesh=_mesh(),
        scratch_types=[
            pltpu.VMEM((EPW,), jnp.int32),
            pltpu.VMEM((3, CG, H), F32),
            pltpu.SemaphoreType.DMA,
            pltpu.SemaphoreType.DMA,
            pltpu.SemaphoreType.DMA,
            pltpu.SemaphoreType.DMA,
            pltpu.SemaphoreType.DMA,
            pltpu.SemaphoreType.DMA,
        ],
    )
    def k(table_hbm, idx_hbm, out_hbm, idx_all, rows, g0, g1, g2, o0, o1, o2):
        gsems = (g0, g1, g2)
        osems = (o0, o1, o2)
        w0 = _wid() * EPW
        pltpu.sync_copy(idx_hbm.at[pl.ds(w0, EPW)], idx_all)

        def gcp(ci, b, n):
            off = pl.multiple_of(ci * CG, 8)
            return pltpu.make_async_copy(
                table_hbm.at[idx_all.at[pl.ds(off, n)]],
                rows.at[b, pl.ds(0, n)], gsems[b])

        def ow_start(ci, b, n):
            off = pl.multiple_of(ci * CG, 8)
            pltpu.async_copy(rows.at[b, pl.ds(0, n)],
                             out_hbm.at[pl.ds(w0 + off, n)], osems[b])

        def ow_wait(b, n):
            pltpu.make_async_copy(rows.at[b, pl.ds(0, n)],
                                  out_hbm.at[pl.ds(0, n)], osems[b]).wait()

        gcp(0, 0, CG).start()

        def outer(i, carry):
            for b in range(3):
                ci = i * 3 + b
                gcp(ci, b, CG).wait()
                nb = (b + 1) % 3

                @pl.when(ci + 1 < NCG)
                def _():
                    @pl.when(ci >= 2)
                    def _():
                        ow_wait(nb, CG)

                    gcp(ci + 1, nb, CG).start()

                ow_start(ci, b, CG)
            return carry

        lax.fori_loop(0, NCG // 3, outer, 0)
        ow_wait(1, CG)
        ow_wait(2, CG)
        ow_wait(0, CG)

        gcp(NCG, 0, TAILG).start()
        gcp(NCG, 0, TAILG).wait()
        ow_start(NCG, 0, TAILG)
        ow_wait(0, TAILG)

    return k


CE2 = 24

S1 = 81920
S2 = E - S1
PW1 = S1 // NW
PW2 = S2 // NW


@functools.cache
def _sc_msg(lo, npw):
    nch = npw // CE2
    tail = npw - nch * CE2
    rows = NW * npw

    @functools.partial(
        pl.kernel,
        out_type=jax.ShapeDtypeStruct((rows, 2 * H), F32),
        mesh=_mesh(),
        scratch_types=[
            pltpu.VMEM((NB * npw,), jnp.int32),
            pltpu.VMEM((2, NB, CE2, 2 * H), F32),
            pltpu.VMEM((2, CE2, H), F32),
            pltpu.VMEM((2, CE2, 2 * H), F32),
            pltpu.SemaphoreType.DMA,
            pltpu.SemaphoreType.DMA,
            pltpu.SemaphoreType.DMA,
            pltpu.SemaphoreType.DMA,
        ],
    )
    def k(tab_hbm, rm_hbm, bgT_hbm, out_hbm, idx_all, gb, rmv, ob,
          gs0, gs1, os0, os1):
        gsems = (gs0, gs1)
        osems = (os0, os1)
        w0l = _wid() * npw
        w0g = lo + w0l
        for j in range(NB):
            pltpu.sync_copy(bgT_hbm.at[pl.ds(j * E + w0g, npw)],
                            idx_all.at[pl.ds(j * npw, npw)])

        def gather_cps(ci, b, n):
            off = pl.multiple_of(ci * CE2, 8)
            cps = [
                pltpu.make_async_copy(
                    tab_hbm.at[idx_all.at[pl.ds(j * npw + off, n)]],
                    gb.at[b, j, pl.ds(0, n)], gsems[b])
                for j in range(NB)
            ]
            cps.append(pltpu.make_async_copy(
                rm_hbm.at[pl.ds(w0g + off, n)],
                rmv.at[b, pl.ds(0, n)], gsems[b]))
            return cps

        def owrite_start(ci, b, n):
            off = pl.multiple_of(ci * CE2, 8)
            pltpu.async_copy(ob.at[b, pl.ds(0, n)],
                             out_hbm.at[pl.ds(w0l + off, n)], osems[b])

        def owrite_wait(b, n):
            pltpu.make_async_copy(ob.at[b, pl.ds(0, n)],
                                  out_hbm.at[pl.ds(0, n)], osems[b]).wait()

        def compute(b, n):
            @plsc.parallel_loop(0, n)
            def edge(e):
                for sl in range(H // 16):
                    o = sl * 16
                    rv = rmv[b, e, pl.ds(o, 16)]
                    accs = jnp.zeros((16,), F32)
                    accg = jnp.zeros((16,), F32)
                    for j in range(NB):
                        hv = gb[b, j, e, pl.ds(o, 16)]
                        uv = gb[b, j, e, pl.ds(H + o, 16)]
                        g = 1.0 / (1.0 + jnp.exp(rv + uv))
                        accs = accs + hv
                        accg = accg + g * hv
                    ob[b, e, pl.ds(o, 16)] = accs
                    ob[b, e, pl.ds(H + o, 16)] = accg

        for cp in gather_cps(0, 0, CE2):
            cp.start()

        def outer(i, carry):
            for b in range(2):
                ci = i * 2 + b
                for cp in gather_cps(ci, b, CE2):
                    cp.wait()

                @pl.when(ci + 1 < nch)
                def _():
                    for cp in gather_cps(ci + 1, 1 - b, CE2):
                        cp.start()

                @pl.when(ci >= 2)
                def _():
                    owrite_wait(b, CE2)

                compute(b, CE2)
                owrite_start(ci, b, CE2)
            return carry

        lax.fori_loop(0, nch // 2, outer, 0)

        if nch % 2 == 1:
            ci = nch - 1
            for cp in gather_cps(ci, 0, CE2):
                cp.wait()
            owrite_wait(0, CE2)
            compute(0, CE2)
            owrite_start(ci, 0, CE2)
            tb = 1
        else:
            tb = 0

        if tail:
            owrite_wait(tb, CE2)
            for cp in gather_cps(nch, tb, tail):
                cp.start()
            for cp in gather_cps(nch, tb, tail):
                cp.wait()
            compute(tb, tail)
            owrite_start(nch, tb, tail)
            owrite_wait(tb, tail)
        owrite_wait(1 - tb, CE2)

    return k


@functools.cache
def _sc_nbr():
    @functools.partial(
        pl.kernel,
        out_type=jax.ShapeDtypeStruct((N, H), F32),
        mesh=_mesh(),
        scratch_types=[
            pltpu.VMEM((NB, RN), jnp.int32),
            pltpu.VMEM((NB, RN, H), F32),
            pltpu.VMEM((RN, H), F32),
            pltpu.SemaphoreType.DMA,
        ],
    )
    def k(h_hbm, agT_hbm, out_hbm, idx_v, gb_v, ob_v, sem):
        w = _wid()
        steps = (NODE_CHUNKS + NW - 1) // NW

        def step(si, carry):
            ci = w + si * NW

            @pl.when(ci < NODE_CHUNKS)
            def _():
                base = pl.multiple_of(ci * RN, 8)
                for j in range(NB):
                    pltpu.sync_copy(agT_hbm.at[pl.ds(j * N + base, RN)],
                                    idx_v.at[j])
                cps = [
                    pltpu.async_copy(h_hbm.at[idx_v.at[j]], gb_v.at[j], sem)
                    for j in range(NB)
                ]
                for cp in cps:
                    cp.wait()

                @plsc.parallel_loop(0, RN)
                def node(e):
                    for sl in range(H // 16):
                        o = sl * 16
                        acc = jnp.zeros((16,), F32)
                        for j in range(NB):
                            acc = acc + gb_v[j, e, pl.ds(o, 16)]
                        ob_v[e, pl.ds(o, 16)] = acc
                pltpu.sync_copy(ob_v, out_hbm.at[pl.ds(base, RN)])

            return carry

        lax.fori_loop(0, steps, step, 0)

    return k


def _dot(a, b):
    return jnp.dot(a, b, preferred_element_type=F32)


def _mask_row0(x, enable=True):
    if not enable:
        return x
    rows = lax.broadcasted_iota(jnp.int32, x.shape, 0)
    first = pl.program_id(0) == 0
    return jnp.where(jnp.logical_and(rows == 0, first), 0.0, x)


def _tc_pre_body(fs_ref, ef_ref, wpre, bz, bh, urT,
                 pz_ref, rm_ref, ph_ref, tab_ref):
    X = jnp.concatenate([fs_ref[...], ef_ref[...]], axis=1)
    P3 = _dot(X, wpre[...])
    pz = P3[:, :H] + bz[...]
    ph = P3[:, 2 * H:] + bh[...]
    pz_ref[...] = pz
    rm_ref[...] = P3[:, H:2 * H]
    ph_ref[...] = ph
    h1 = jax.nn.sigmoid(pz) * jnp.tanh(ph)
    h1 = _mask_row0(h1)
    tab_ref[:, :H] = h1
    tab_ref[:, H:] = _dot(h1, -urT[...])


@functools.cache
def _tc_pre():
    rspec = lambda w: pl.BlockSpec((RB, w), lambda i: (i, 0))
    wspec = pl.BlockSpec((H, H), lambda i: (0, 0))
    pspec = pl.BlockSpec((H + EF, 3 * H), lambda i: (0, 0))
    bspec = pl.BlockSpec((1, H), lambda i: (0, 0))
    return pl.pallas_call(
        _tc_pre_body,
        grid=(E // RB,),
        in_specs=[rspec(H), rspec(EF), pspec, bspec, bspec, wspec],
        out_specs=[rspec(H), rspec(H), rspec(H), rspec(2 * H)],
        out_shape=[jax.ShapeDtypeStruct((E, H), F32)] * 3
        + [jax.ShapeDtypeStruct((E, 2 * H), F32)],
    )


def _tc_gru_body(sum_ref, pz_ref, ph_ref, wzp, urT, out_ref, *, last,
                 mask):
    s_h = sum_ref[:, :H]
    zp = _dot(sum_ref[...], wzp[...])
    z = jax.nn.sigmoid(pz_ref[...] + zp[:, :H])
    p = jnp.tanh(ph_ref[...] + zp[:, H:])
    h = (1.0 - z) * s_h + z * p
    h = _mask_row0(h, mask)
    if last:
        out_ref[...] = h
    else:
        out_ref[:, :H] = h
        out_ref[:, H:] = _dot(h, -urT[...])


@functools.cache
def _tc_gru(last, lo, rows):
    blk0 = lo // RB
    rspec = lambda w: pl.BlockSpec((RB, w), lambda i: (i, 0))
    gspec = lambda w: pl.BlockSpec((RB, w), lambda i: (i + blk0, 0))
    wspec = pl.BlockSpec((H, H), lambda i: (0, 0))
    zspec = pl.BlockSpec((2 * H, 2 * H), lambda i: (0, 0))
    ow = H if last else 2 * H
    specs = [rspec(2 * H), gspec(H), gspec(H), zspec, wspec]
    return pl.pallas_call(
        functools.partial(_tc_gru_body, last=last, mask=(lo == 0)),
        grid=(rows // RB,),
        in_specs=specs,
        out_specs=rspec(ow),
        out_shape=jax.ShapeDtypeStruct((rows, ow), F32),
    )


def _tc_out_body(fn_ref, nei_ref, wo12, bo, hatom_ref, hmol_ref):
    X = jnp.concatenate([fn_ref[0], nei_ref[0]], axis=1)
    x = _dot(X, wo12[...]) + bo[...]
    x = jnp.maximum(x, 0.0)
    x = _mask_row0(x)
    hatom_ref[0] = x
    hmol_ref[0] = jnp.sum(x, axis=0, keepdims=True)


@functools.cache
def _tc_out():
    rspec = pl.BlockSpec((1, MOLSZ, H), lambda i: (i, 0, 0))
    wspec = pl.BlockSpec((2 * H, H), lambda i: (0, 0))
    bspec = pl.BlockSpec((1, H), lambda i: (0, 0))
    return pl.pallas_call(
        _tc_out_body,
        grid=(NMOL,),
        in_specs=[rspec, rspec, wspec, bspec],
        out_specs=[rspec, pl.BlockSpec((1, 1, H), lambda i: (i, 0, 0))],
        out_shape=[jax.ShapeDtypeStruct((NMOL, MOLSZ, H), F32),
                   jax.ShapeDtypeStruct((NMOL, 1, H), F32)],
    )


def kernel(fnode, fmess, agraph, bgraph, atom_scope, W_z, b_z, W_r, U_r,
           W_h, b_h, W_o, b_o):
    src = fmess[:, 0].astype(jnp.int32)
    efeat = fmess[:, 2:]
    bgT = bgraph.T.reshape(-1)
    agT = agraph.T.reshape(-1)

    wz2 = W_z[:, H + EF:].T
    wh2 = W_h[:, H + EF:].T
    wpre = jnp.concatenate(
        [W_z[:, :H + EF].T, -W_r.T, W_h[:, :H + EF].T], axis=1)
    zeroH = jnp.zeros((H, H), F32)
    wzp = jnp.concatenate(
        [jnp.concatenate([wz2, zeroH], axis=1),
         jnp.concatenate([zeroH, wh2], axis=1)], axis=0)
    wo12 = W_o.T
    urT = U_r.T
    bz = b_z.reshape(1, H)
    bh = b_h.reshape(1, H)
    bo = b_o.reshape(1, H)

    fsrc = _sc_gather_rows()(fnode, src)
    pz, rm, ph, tab = _tc_pre()(fsrc, efeat, wpre, bz, bh, urT)
    sums = _sc_msg(0, EPW)(tab, rm, bgT)
    tab = _tc_gru(False, 0, E)(sums, pz, ph, wzp, urT)
    sums = _sc_msg(0, EPW)(tab, rm, bgT)
    h = _tc_gru(True, 0, E)(sums, pz, ph, wzp, urT)
    nei = _sc_nbr()(h, agT)
    hatom3, hmol3 = _tc_out()(fnode.reshape(NMOL, MOLSZ, H),
                              nei.reshape(NMOL, MOLSZ, H), wo12, bo)
    return (hmol3.reshape(NMOL, H), hatom3.reshape(N, H))


# speedup vs baseline: 1.1968x; 1.0280x over previous
import functools

import jax
import jax.numpy as jnp
from jax import lax
from jax.experimental import pallas as pl
from jax.experimental.pallas import tpu as pltpu
from jax.experimental.pallas import tpu_sc as plsc

E = 160000
N = 10000
H = 128
EF = 16
NB = 6
NMOL = 100
MOLSZ = 100

NC = 2
NS = 16
NW = NC * NS
EPW = E // NW
CE = 40
NCHUNK = EPW // CE
RN = 40
NODE_CHUNKS = N // RN
RB = 3200
F32 = jnp.float32


def _mesh():
    return plsc.VectorSubcoreMesh(
        core_axis_name="c", subcore_axis_name="s", num_cores=NC, num_subcores=NS
    )


def _wid():
    return lax.axis_index("s") * NC + lax.axis_index("c")


def _sigmoid16(x):
    return 1.0 / (1.0 + jnp.exp(-x))


CG = 128
NCG = EPW // CG
TAILG = EPW - NCG * CG


@functools.cache
def _sc_gather_rows():
    @functools.partial(
        pl.kernel,
        out_type=jax.ShapeDtypeStruct((E, H), F32),
        mesh=_mesh(),
        scratch_types=[
            pltpu.VMEM((EPW,), jnp.int32),
            pltpu.VMEM((3, CG, H), F32),
            pltpu.SemaphoreType.DMA,
            pltpu.SemaphoreType.DMA,
            pltpu.SemaphoreType.DMA,
            pltpu.SemaphoreType.DMA,
            pltpu.SemaphoreType.DMA,
            pltpu.SemaphoreType.DMA,
        ],
    )
    def k(table_hbm, idx_hbm, out_hbm, idx_all, rows, g0, g1, g2, o0, o1, o2):
        gsems = (g0, g1, g2)
        osems = (o0, o1, o2)
        w0 = _wid() * EPW
        pltpu.sync_copy(idx_hbm.at[pl.ds(w0, EPW)], idx_all)

        def gcp(ci, b, n):
            off = pl.multiple_of(ci * CG, 8)
            return pltpu.make_async_copy(
                table_hbm.at[idx_all.at[pl.ds(off, n)]],
                rows.at[b, pl.ds(0, n)], gsems[b])

        def ow_start(ci, b, n):
            off = pl.multiple_of(ci * CG, 8)
            pltpu.async_copy(rows.at[b, pl.ds(0, n)],
                             out_hbm.at[pl.ds(w0 + off, n)], osems[b])

        def ow_wait(b, n):
            pltpu.make_async_copy(rows.at[b, pl.ds(0, n)],
                                  out_hbm.at[pl.ds(0, n)], osems[b]).wait()

        gcp(0, 0, CG).start()

        def outer(i, carry):
            for b in range(3):
                ci = i * 3 + b
                gcp(ci, b, CG).wait()
                nb = (b + 1) % 3

                @pl.when(ci + 1 < NCG)
                def _():
                    @pl.when(ci >= 2)
                    def _():
                        ow_wait(nb, CG)

                    gcp(ci + 1, nb, CG).start()

                ow_start(ci, b, CG)
            return carry

        lax.fori_loop(0, NCG // 3, outer, 0)
        ow_wait(1, CG)
        ow_wait(2, CG)
        ow_wait(0, CG)

        gcp(NCG, 0, TAILG).start()
        gcp(NCG, 0, TAILG).wait()
        ow_start(NCG, 0, TAILG)
        ow_wait(0, TAILG)

    return k


CE2 = 24

S1 = 81920
S2 = E - S1
PW1 = S1 // NW
PW2 = S2 // NW


@functools.cache
def _sc_msg(lo, npw):
    nch = npw // CE2
    tail = npw - nch * CE2
    rows = NW * npw

    @functools.partial(
        pl.kernel,
        out_type=jax.ShapeDtypeStruct((rows, 2 * H), F32),
        mesh=_mesh(),
        scratch_types=[
            pltpu.VMEM((NB * npw,), jnp.int32),
            pltpu.VMEM((2, NB, CE2, 2 * H), F32),
            pltpu.VMEM((2, CE2, H), F32),
            pltpu.VMEM((2, CE2, 2 * H), F32),
            pltpu.SemaphoreType.DMA,
            pltpu.SemaphoreType.DMA,
            pltpu.SemaphoreType.DMA,
            pltpu.SemaphoreType.DMA,
        ],
    )
    def k(tab_hbm, rm_hbm, bgT_hbm, out_hbm, idx_all, gb, rmv, ob,
          gs0, gs1, os0, os1):
        gsems = (gs0, gs1)
        osems = (os0, os1)
        w0l = _wid() * npw
        w0g = lo + w0l
        for j in range(NB):
            pltpu.sync_copy(bgT_hbm.at[pl.ds(j * E + w0g, npw)],
                            idx_all.at[pl.ds(j * npw, npw)])

        def gather_cps(ci, b, n):
            off = pl.multiple_of(ci * CE2, 8)
            cps = [
                pltpu.make_async_copy(
                    tab_hbm.at[idx_all.at[pl.ds(j * npw + off, n)]],
                    gb.at[b, j, pl.ds(0, n)], gsems[b])
                for j in range(NB)
            ]
            cps.append(pltpu.make_async_copy(
                rm_hbm.at[pl.ds(w0g + off, n)],
                rmv.at[b, pl.ds(0, n)], gsems[b]))
            return cps

        def owrite_start(ci, b, n):
            off = pl.multiple_of(ci * CE2, 8)
            pltpu.async_copy(ob.at[b, pl.ds(0, n)],
                             out_hbm.at[pl.ds(w0l + off, n)], osems[b])

        def owrite_wait(b, n):
            pltpu.make_async_copy(ob.at[b, pl.ds(0, n)],
                                  out_hbm.at[pl.ds(0, n)], osems[b]).wait()

        def compute(b, n):
            @plsc.parallel_loop(0, n)
            def edge(e):
                for sl in range(H // 16):
                    o = sl * 16
                    rv = rmv[b, e, pl.ds(o, 16)]
                    accs = jnp.zeros((16,), F32)
                    accg = jnp.zeros((16,), F32)
                    for j in range(NB):
                        hv = gb[b, j, e, pl.ds(o, 16)]
                        uv = gb[b, j, e, pl.ds(H + o, 16)]
                        g = 1.0 / (1.0 + jnp.exp(rv + uv))
                        accs = accs + hv
                        accg = accg + g * hv
                    ob[b, e, pl.ds(o, 16)] = accs
                    ob[b, e, pl.ds(H + o, 16)] = accg

        for cp in gather_cps(0, 0, CE2):
            cp.start()

        def outer(i, carry):
            for b in range(2):
                ci = i * 2 + b
                for cp in gather_cps(ci, b, CE2):
                    cp.wait()

                @pl.when(ci + 1 < nch)
                def _():
                    for cp in gather_cps(ci + 1, 1 - b, CE2):
                        cp.start()

                @pl.when(ci >= 2)
                def _():
                    owrite_wait(b, CE2)

                compute(b, CE2)
                owrite_start(ci, b, CE2)
            return carry

        lax.fori_loop(0, nch // 2, outer, 0)

        if nch % 2 == 1:
            ci = nch - 1
            for cp in gather_cps(ci, 0, CE2):
                cp.wait()
            owrite_wait(0, CE2)
            compute(0, CE2)
            owrite_start(ci, 0, CE2)
            tb = 1
        else:
            tb = 0

        if tail:
            owrite_wait(tb, CE2)
            for cp in gather_cps(nch, tb, tail):
                cp.start()
            for cp in gather_cps(nch, tb, tail):
                cp.wait()
            compute(tb, tail)
            owrite_start(nch, tb, tail)
            owrite_wait(tb, tail)
        owrite_wait(1 - tb, CE2)

    return k


@functools.cache
def _sc_nbr():
    @functools.partial(
        pl.kernel,
        out_type=jax.ShapeDtypeStruct((N, H), F32),
        mesh=_mesh(),
        scratch_types=[
            pltpu.VMEM((NB, RN), jnp.int32),
            pltpu.VMEM((NB, RN, H), F32),
            pltpu.VMEM((RN, H), F32),
            pltpu.SemaphoreType.DMA,
        ],
    )
    def k(h_hbm, agT_hbm, out_hbm, idx_v, gb_v, ob_v, sem):
        w = _wid()
        steps = (NODE_CHUNKS + NW - 1) // NW

        def step(si, carry):
            ci = w + si * NW

            @pl.when(ci < NODE_CHUNKS)
            def _():
                base = pl.multiple_of(ci * RN, 8)
                for j in range(NB):
                    pltpu.sync_copy(agT_hbm.at[pl.ds(j * N + base, RN)],
                                    idx_v.at[j])
                cps = [
                    pltpu.async_copy(h_hbm.at[idx_v.at[j]], gb_v.at[j], sem)
                    for j in range(NB)
                ]
                for cp in cps:
                    cp.wait()

                @plsc.parallel_loop(0, RN)
                def node(e):
                    for sl in range(H // 16):
                        o = sl * 16
                        acc = jnp.zeros((16,), F32)
                        for j in range(NB):
                            acc = acc + gb_v[j, e, pl.ds(o, 16)]
                        ob_v[e, pl.ds(o, 16)] = acc
                pltpu.sync_copy(ob_v, out_hbm.at[pl.ds(base, RN)])

            return carry

        lax.fori_loop(0, steps, step, 0)

    return k


def _dot(a, b):
    return jnp.dot(a, b, preferred_element_type=F32)


def _mask_row0(x, enable=True):
    if not enable:
        return x
    rows = lax.broadcasted_iota(jnp.int32, x.shape, 0)
    first = pl.program_id(0) == 0
    return jnp.where(jnp.logical_and(rows == 0, first), 0.0, x)


def _tc_pre_body(fs_ref, ef_ref, wpre, bz, bh, urT,
                 pz_ref, rm_ref, ph_ref, tab_ref):
    X = jnp.concatenate([fs_ref[...], ef_ref[...]], axis=1)
    P3 = _dot(X, wpre[...])
    pz = P3[:, :H] + bz[...]
    ph = P3[:, 2 * H:] + bh[...]
    pz_ref[...] = pz
    rm_ref[...] = P3[:, H:2 * H]
    ph_ref[...] = ph
    h1 = jax.nn.sigmoid(pz) * jnp.tanh(ph)
    h1 = _mask_row0(h1)
    tab_ref[:, :H] = h1
    tab_ref[:, H:] = _dot(h1, -urT[...])


@functools.cache
def _tc_pre():
    rspec = lambda w: pl.BlockSpec((RB, w), lambda i: (i, 0))
    wspec = pl.BlockSpec((H, H), lambda i: (0, 0))
    pspec = pl.BlockSpec((H + EF, 3 * H), lambda i: (0, 0))
    bspec = pl.BlockSpec((1, H), lambda i: (0, 0))
    return pl.pallas_call(
        _tc_pre_body,
        grid=(E // RB,),
        in_specs=[rspec(H), rspec(EF), pspec, bspec, bspec, wspec],
        out_specs=[rspec(H), rspec(H), rspec(H), rspec(2 * H)],
        out_shape=[jax.ShapeDtypeStruct((E, H), F32)] * 3
        + [jax.ShapeDtypeStruct((E, 2 * H), F32)],
    )


def _tc_gru_body(sum_ref, pz_ref, ph_ref, wzp, urT, out_ref, *, last,
                 mask):
    s_h = sum_ref[:, :H]
    zp = _dot(sum_ref[...], wzp[...])
    z = jax.nn.sigmoid(pz_ref[...] + zp[:, :H])
    p = jnp.tanh(ph_ref[...] + zp[:, H:])
    h = (1.0 - z) * s_h + z * p
    h = _mask_row0(h, mask)
    if last:
        out_ref[...] = h
    else:
        out_ref[:, :H] = h
        out_ref[:, H:] = _dot(h, -urT[...])


@functools.cache
def _tc_gru(last, lo, rows):
    blk0 = lo // RB
    rspec = lambda w: pl.BlockSpec((RB, w), lambda i: (i, 0))
    gspec = lambda w: pl.BlockSpec((RB, w), lambda i: (i + blk0, 0))
    wspec = pl.BlockSpec((H, H), lambda i: (0, 0))
    zspec = pl.BlockSpec((2 * H, 2 * H), lambda i: (0, 0))
    ow = H if last else 2 * H
    specs = [rspec(2 * H), gspec(H), gspec(H), zspec, wspec]
    return pl.pallas_call(
        functools.partial(_tc_gru_body, last=last, mask=(lo == 0)),
        grid=(rows // RB,),
        in_specs=specs,
        out_specs=rspec(ow),
        out_shape=jax.ShapeDtypeStruct((rows, ow), F32),
    )


def _tc_out_body(fn_ref, nei_ref, wo12, bo, hatom_ref, hmol_ref):
    X = jnp.concatenate([fn_ref[0], nei_ref[0]], axis=1)
    x = _dot(X, wo12[...]) + bo[...]
    x = jnp.maximum(x, 0.0)
    x = _mask_row0(x)
    hatom_ref[0] = x
    hmol_ref[0] = jnp.sum(x, axis=0, keepdims=True)


@functools.cache
def _tc_out():
    rspec = pl.BlockSpec((1, MOLSZ, H), lambda i: (i, 0, 0))
    wspec = pl.BlockSpec((2 * H, H), lambda i: (0, 0))
    bspec = pl.BlockSpec((1, H), lambda i: (0, 0))
    return pl.pallas_call(
        _tc_out_body,
        grid=(NMOL,),
        in_specs=[rspec, rspec, wspec, bspec],
        out_specs=[rspec, pl.BlockSpec((1, 1, H), lambda i: (i, 0, 0))],
        out_shape=[jax.ShapeDtypeStruct((NMOL, MOLSZ, H), F32),
                   jax.ShapeDtypeStruct((NMOL, 1, H), F32)],
    )


def kernel(fnode, fmess, agraph, bgraph, atom_scope, W_z, b_z, W_r, U_r,
           W_h, b_h, W_o, b_o):
    src = fmess[:, 0].astype(jnp.int32)
    efeat = fmess[:, 2:]
    bgT = bgraph.T.reshape(-1)
    agT = agraph.T.reshape(-1)

    wz2 = W_z[:, H + EF:].T
    wh2 = W_h[:, H + EF:].T
    wpre = jnp.concatenate(
        [W_z[:, :H + EF].T, -W_r.T, W_h[:, :H + EF].T], axis=1)
    zeroH = jnp.zeros((H, H), F32)
    wzp = jnp.concatenate(
        [jnp.concatenate([wz2, zeroH], axis=1),
         jnp.concatenate([zeroH, wh2], axis=1)], axis=0)
    wo12 = W_o.T
    urT = U_r.T
    bz = b_z.reshape(1, H)
    bh = b_h.reshape(1, H)
    bo = b_o.reshape(1, H)

    fsrc = _sc_gather_rows()(fnode, src)
    pz, rm, ph, tab = _tc_pre()(fsrc, efeat, wpre, bz, bh, urT)
    sums = _sc_msg(0, EPW)(tab, rm, bgT)
    tab = _tc_gru(False, 0, E)(sums, pz, ph, wzp, urT)
    sums = _sc_msg(0, EPW)(tab, rm, bgT)
    h = _tc_gru(True, 0, E)(sums, pz, ph, wzp, urT)
    nei = _sc_nbr()(h, agT)
    hatom3, hmol3 = _tc_out()(fnode.reshape(NMOL, MOLSZ, H),
                              nei.reshape(NMOL, MOLSZ, H), wo12, bo)
    return (hmol3.reshape(NMOL, H), hatom3.reshape(N, H))


# speedup vs baseline: 1.2015x; 1.0039x over previous
import functools

import jax
import jax.numpy as jnp
from jax import lax
from jax.experimental import pallas as pl
from jax.experimental.pallas import tpu as pltpu
from jax.experimental.pallas import tpu_sc as plsc

E = 160000
N = 10000
H = 128
EF = 16
NB = 6
NMOL = 100
MOLSZ = 100

NC = 2
NS = 16
NW = NC * NS
EPW = E // NW
CE = 40
NCHUNK = EPW // CE
RN = 40
NODE_CHUNKS = N // RN
RB = 6400
F32 = jnp.float32


def _mesh():
    return plsc.VectorSubcoreMesh(
        core_axis_name="c", subcore_axis_name="s", num_cores=NC, num_subcores=NS
    )


def _wid():
    return lax.axis_index("s") * NC + lax.axis_index("c")


def _sigmoid16(x):
    return 1.0 / (1.0 + jnp.exp(-x))


CG = 128
NCG = EPW // CG
TAILG = EPW - NCG * CG


@functools.cache
def _sc_gather_rows():
    @functools.partial(
        pl.kernel,
        out_type=jax.ShapeDtypeStruct((E, H), F32),
        mesh=_mesh(),
        scratch_types=[
            pltpu.VMEM((EPW,), jnp.int32),
            pltpu.VMEM((3, CG, H), F32),
            pltpu.SemaphoreType.DMA,
            pltpu.SemaphoreType.DMA,
            pltpu.SemaphoreType.DMA,
            pltpu.SemaphoreType.DMA,
            pltpu.SemaphoreType.DMA,
            pltpu.SemaphoreType.DMA,
        ],
    )
    def k(table_hbm, idx_hbm, out_hbm, idx_all, rows, g0, g1, g2, o0, o1, o2):
        gsems = (g0, g1, g2)
        osems = (o0, o1, o2)
        w0 = _wid() * EPW
        pltpu.sync_copy(idx_hbm.at[pl.ds(w0, EPW)], idx_all)

        def gcp(ci, b, n):
            off = pl.multiple_of(ci * CG, 8)
            return pltpu.make_async_copy(
                table_hbm.at[idx_all.at[pl.ds(off, n)]],
                rows.at[b, pl.ds(0, n)], gsems[b])

        def ow_start(ci, b, n):
            off = pl.multiple_of(ci * CG, 8)
            pltpu.async_copy(rows.at[b, pl.ds(0, n)],
                             out_hbm.at[pl.ds(w0 + off, n)], osems[b])

        def ow_wait(b, n):
            pltpu.make_async_copy(rows.at[b, pl.ds(0, n)],
                                  out_hbm.at[pl.ds(0, n)], osems[b]).wait()

        gcp(0, 0, CG).start()

        def outer(i, carry):
            for b in range(3):
                ci = i * 3 + b
                gcp(ci, b, CG).wait()
                nb = (b + 1) % 3

                @pl.when(ci + 1 < NCG)
                def _():
                    @pl.when(ci >= 2)
                    def _():
                        ow_wait(nb, CG)

                    gcp(ci + 1, nb, CG).start()

                ow_start(ci, b, CG)
            return carry

        lax.fori_loop(0, NCG // 3, outer, 0)
        ow_wait(1, CG)
        ow_wait(2, CG)
        ow_wait(0, CG)

        gcp(NCG, 0, TAILG).start()
        gcp(NCG, 0, TAILG).wait()
        ow_start(NCG, 0, TAILG)
        ow_wait(0, TAILG)

    return k


CE2 = 24

S1 = 81920
S2 = E - S1
PW1 = S1 // NW
PW2 = S2 // NW


@functools.cache
def _sc_msg(lo, npw):
    nch = npw // CE2
    tail = npw - nch * CE2
    rows = NW * npw

    @functools.partial(
        pl.kernel,
        out_type=jax.ShapeDtypeStruct((rows, 2 * H), F32),
        mesh=_mesh(),
        scratch_types=[
            pltpu.VMEM((NB * npw,), jnp.int32),
            pltpu.VMEM((2, NB, CE2, 2 * H), F32),
            pltpu.VMEM((2, CE2, H), F32),
            pltpu.VMEM((2, CE2, 2 * H), F32),
            pltpu.SemaphoreType.DMA,
            pltpu.SemaphoreType.DMA,
            pltpu.SemaphoreType.DMA,
            pltpu.SemaphoreType.DMA,
        ],
    )
    def k(tab_hbm, rm_hbm, bgT_hbm, out_hbm, idx_all, gb, rmv, ob,
          gs0, gs1, os0, os1):
        gsems = (gs0, gs1)
        osems = (os0, os1)
        w0l = _wid() * npw
        w0g = lo + w0l
        for j in range(NB):
            pltpu.sync_copy(bgT_hbm.at[pl.ds(j * E + w0g, npw)],
                            idx_all.at[pl.ds(j * npw, npw)])

        def gather_cps(ci, b, n):
            off = pl.multiple_of(ci * CE2, 8)
            cps = [
                pltpu.make_async_copy(
                    tab_hbm.at[idx_all.at[pl.ds(j * npw + off, n)]],
                    gb.at[b, j, pl.ds(0, n)], gsems[b])
                for j in range(NB)
            ]
            cps.append(pltpu.make_async_copy(
                rm_hbm.at[pl.ds(w0g + off, n)],
                rmv.at[b, pl.ds(0, n)], gsems[b]))
            return cps

        def owrite_start(ci, b, n):
            off = pl.multiple_of(ci * CE2, 8)
            pltpu.async_copy(ob.at[b, pl.ds(0, n)],
                             out_hbm.at[pl.ds(w0l + off, n)], osems[b])

        def owrite_wait(b, n):
            pltpu.make_async_copy(ob.at[b, pl.ds(0, n)],
                                  out_hbm.at[pl.ds(0, n)], osems[b]).wait()

        def compute(b, n):
            @plsc.parallel_loop(0, n)
            def edge(e):
                for sl in range(H // 16):
                    o = sl * 16
                    rv = rmv[b, e, pl.ds(o, 16)]
                    accs = jnp.zeros((16,), F32)
                    accg = jnp.zeros((16,), F32)
                    for j in range(NB):
                        hv = gb[b, j, e, pl.ds(o, 16)]
                        uv = gb[b, j, e, pl.ds(H + o, 16)]
                        g = 1.0 / (1.0 + jnp.exp(rv + uv))
                        accs = accs + hv
                        accg = accg + g * hv
                    ob[b, e, pl.ds(o, 16)] = accs
                    ob[b, e, pl.ds(H + o, 16)] = accg

        for cp in gather_cps(0, 0, CE2):
            cp.start()

        def outer(i, carry):
            for b in range(2):
                ci = i * 2 + b
                for cp in gather_cps(ci, b, CE2):
                    cp.wait()

                @pl.when(ci + 1 < nch)
                def _():
                    for cp in gather_cps(ci + 1, 1 - b, CE2):
                        cp.start()

                @pl.when(ci >= 2)
                def _():
                    owrite_wait(b, CE2)

                compute(b, CE2)
                owrite_start(ci, b, CE2)
            return carry

        lax.fori_loop(0, nch // 2, outer, 0)

        if nch % 2 == 1:
            ci = nch - 1
            for cp in gather_cps(ci, 0, CE2):
                cp.wait()
            owrite_wait(0, CE2)
            compute(0, CE2)
            owrite_start(ci, 0, CE2)
            tb = 1
        else:
            tb = 0

        if tail:
            owrite_wait(tb, CE2)
            for cp in gather_cps(nch, tb, tail):
                cp.start()
            for cp in gather_cps(nch, tb, tail):
                cp.wait()
            compute(tb, tail)
            owrite_start(nch, tb, tail)
            owrite_wait(tb, tail)
        owrite_wait(1 - tb, CE2)

    return k


@functools.cache
def _sc_nbr():
    @functools.partial(
        pl.kernel,
        out_type=jax.ShapeDtypeStruct((N, H), F32),
        mesh=_mesh(),
        scratch_types=[
            pltpu.VMEM((NB, RN), jnp.int32),
            pltpu.VMEM((NB, RN, H), F32),
            pltpu.VMEM((RN, H), F32),
            pltpu.SemaphoreType.DMA,
        ],
    )
    def k(h_hbm, agT_hbm, out_hbm, idx_v, gb_v, ob_v, sem):
        w = _wid()
        steps = (NODE_CHUNKS + NW - 1) // NW

        def step(si, carry):
            ci = w + si * NW

            @pl.when(ci < NODE_CHUNKS)
            def _():
                base = pl.multiple_of(ci * RN, 8)
                for j in range(NB):
                    pltpu.sync_copy(agT_hbm.at[pl.ds(j * N + base, RN)],
                                    idx_v.at[j])
                cps = [
                    pltpu.async_copy(h_hbm.at[idx_v.at[j]], gb_v.at[j], sem)
                    for j in range(NB)
                ]
                for cp in cps:
                    cp.wait()

                @plsc.parallel_loop(0, RN)
                def node(e):
                    for sl in range(H // 16):
                        o = sl * 16
                        acc = jnp.zeros((16,), F32)
                        for j in range(NB):
                            acc = acc + gb_v[j, e, pl.ds(o, 16)]
                        ob_v[e, pl.ds(o, 16)] = acc
                pltpu.sync_copy(ob_v, out_hbm.at[pl.ds(base, RN)])

            return carry

        lax.fori_loop(0, steps, step, 0)

    return k


def _dot(a, b):
    return jnp.dot(a, b, preferred_element_type=F32)


def _mask_row0(x, enable=True):
    if not enable:
        return x
    rows = lax.broadcasted_iota(jnp.int32, x.shape, 0)
    first = pl.program_id(0) == 0
    return jnp.where(jnp.logical_and(rows == 0, first), 0.0, x)


def _tc_pre_body(fs_ref, ef_ref, wpre, bz, bh, urT,
                 pz_ref, rm_ref, ph_ref, tab_ref):
    X = jnp.concatenate([fs_ref[...], ef_ref[...]], axis=1)
    P3 = _dot(X, wpre[...])
    pz = P3[:, :H] + bz[...]
    ph = P3[:, 2 * H:] + bh[...]
    pz_ref[...] = pz
    rm_ref[...] = P3[:, H:2 * H]
    ph_ref[...] = ph
    h1 = jax.nn.sigmoid(pz) * jnp.tanh(ph)
    h1 = _mask_row0(h1)
    tab_ref[:, :H] = h1
    tab_ref[:, H:] = _dot(h1, -urT[...])


@functools.cache
def _tc_pre():
    rspec = lambda w: pl.BlockSpec((RB, w), lambda i: (i, 0))
    wspec = pl.BlockSpec((H, H), lambda i: (0, 0))
    pspec = pl.BlockSpec((H + EF, 3 * H), lambda i: (0, 0))
    bspec = pl.BlockSpec((1, H), lambda i: (0, 0))
    return pl.pallas_call(
        _tc_pre_body,
        grid=(E // RB,),
        in_specs=[rspec(H), rspec(EF), pspec, bspec, bspec, wspec],
        out_specs=[rspec(H), rspec(H), rspec(H), rspec(2 * H)],
        out_shape=[jax.ShapeDtypeStruct((E, H), F32)] * 3
        + [jax.ShapeDtypeStruct((E, 2 * H), F32)],
    )


def _tc_gru_body(sum_ref, pz_ref, ph_ref, wzp, urT, out_ref, *, last,
                 mask):
    s_h = sum_ref[:, :H]
    zp = _dot(sum_ref[...], wzp[...])
    z = jax.nn.sigmoid(pz_ref[...] + zp[:, :H])
    p = jnp.tanh(ph_ref[...] + zp[:, H:])
    h = (1.0 - z) * s_h + z * p
    h = _mask_row0(h, mask)
    if last:
        out_ref[...] = h
    else:
        out_ref[:, :H] = h
        out_ref[:, H:] = _dot(h, -urT[...])


@functools.cache
def _tc_gru(last, lo, rows):
    blk0 = lo // RB
    rspec = lambda w: pl.BlockSpec((RB, w), lambda i: (i, 0))
    gspec = lambda w: pl.BlockSpec((RB, w), lambda i: (i + blk0, 0))
    wspec = pl.BlockSpec((H, H), lambda i: (0, 0))
    zspec = pl.BlockSpec((2 * H, 2 * H), lambda i: (0, 0))
    ow = H if last else 2 * H
    specs = [rspec(2 * H), gspec(H), gspec(H), zspec, wspec]
    return pl.pallas_call(
        functools.partial(_tc_gru_body, last=last, mask=(lo == 0)),
        grid=(rows // RB,),
        in_specs=specs,
        out_specs=rspec(ow),
        out_shape=jax.ShapeDtypeStruct((rows, ow), F32),
    )


def _tc_out_body(fn_ref, nei_ref, wo12, bo, hatom_ref, hmol_ref):
    X = jnp.concatenate([fn_ref[0], nei_ref[0]], axis=1)
    x = _dot(X, wo12[...]) + bo[...]
    x = jnp.maximum(x, 0.0)
    x = _mask_row0(x)
    hatom_ref[0] = x
    hmol_ref[0] = jnp.sum(x, axis=0, keepdims=True)


@functools.cache
def _tc_out():
    rspec = pl.BlockSpec((1, MOLSZ, H), lambda i: (i, 0, 0))
    wspec = pl.BlockSpec((2 * H, H), lambda i: (0, 0))
    bspec = pl.BlockSpec((1, H), lambda i: (0, 0))
    return pl.pallas_call(
        _tc_out_body,
        grid=(NMOL,),
        in_specs=[rspec, rspec, wspec, bspec],
        out_specs=[rspec, pl.BlockSpec((1, 1, H), lambda i: (i, 0, 0))],
        out_shape=[jax.ShapeDtypeStruct((NMOL, MOLSZ, H), F32),
                   jax.ShapeDtypeStruct((NMOL, 1, H), F32)],
    )


def kernel(fnode, fmess, agraph, bgraph, atom_scope, W_z, b_z, W_r, U_r,
           W_h, b_h, W_o, b_o):
    src = fmess[:, 0].astype(jnp.int32)
    efeat = fmess[:, 2:]
    bgT = bgraph.T.reshape(-1)
    agT = agraph.T.reshape(-1)

    wz2 = W_z[:, H + EF:].T
    wh2 = W_h[:, H + EF:].T
    wpre = jnp.concatenate(
        [W_z[:, :H + EF].T, -W_r.T, W_h[:, :H + EF].T], axis=1)
    zeroH = jnp.zeros((H, H), F32)
    wzp = jnp.concatenate(
        [jnp.concatenate([wz2, zeroH], axis=1),
         jnp.concatenate([zeroH, wh2], axis=1)], axis=0)
    wo12 = W_o.T
    urT = U_r.T
    bz = b_z.reshape(1, H)
    bh = b_h.reshape(1, H)
    bo = b_o.reshape(1, H)

    fsrc = _sc_gather_rows()(fnode, src)
    pz, rm, ph, tab = _tc_pre()(fsrc, efeat, wpre, bz, bh, urT)
    sums = _sc_msg(0, EPW)(tab, rm, bgT)
    tab = _tc_gru(False, 0, E)(sums, pz, ph, wzp, urT)
    sums = _sc_msg(0, EPW)(tab, rm, bgT)
    h = _tc_gru(True, 0, E)(sums, pz, ph, wzp, urT)
    nei = _sc_nbr()(h, agT)
    hatom3, hmol3 = _tc_out()(fnode.reshape(NMOL, MOLSZ, H),
                              nei.reshape(NMOL, MOLSZ, H), wo12, bo)
    return (hmol3.reshape(NMOL, H), hatom3.reshape(N, H))
